# baseline (device time: 192084 ns/iter reference)
import jax
import jax.numpy as jnp
from jax import lax
from jax.experimental import pallas as pl
from jax.experimental.pallas import tpu as pltpu

N_DEV = 4
B, SQ, D = 4, 256, 1024
HQ, HKV, DH = 8, 2, 128
G = HQ // HKV
SCALE = 0.08838834764831843
NIDX = B * HQ
O_ROWS = NIDX * SQ
L_ROW0 = O_ROWS
ROWS = O_ROWS + SQ


def kernel(x, Wq, Wo, K_ext, V_ext):
    def body(x_ref, wq_ref, wo_ref, k_ref, v_ref, out_ref,
             comm, att, send_sems, recv_sems):
        my = lax.axis_index("i")
        left = (my - 1) % N_DEV
        right = (my + 1) % N_DEV

        barrier_sem = pltpu.get_barrier_semaphore()
        for nbr in (left, right):
            pl.semaphore_signal(barrier_sem, inc=1, device_id=(nbr,),
                                device_id_type=pl.DeviceIdType.MESH)
        pl.semaphore_wait(barrier_sem, 2)

        q = jnp.dot(x_ref[:].reshape(B * SQ, D), wq_ref[:],
                    preferred_element_type=jnp.float32)

        for b in range(B):
            for h in range(HQ):
                g = h // G
                idx = b * HQ + h
                qbh = q[b * SQ:(b + 1) * SQ, h * DH:(h + 1) * DH]
                kb = k_ref[b, :, g, :]
                vb = v_ref[b, :, g, :]
                s = lax.dot_general(
                    qbh, kb, (((1,), (1,)), ((), ())),
                    preferred_element_type=jnp.float32) * SCALE
                p = jnp.exp(s)
                l = jnp.sum(p, axis=1, keepdims=True)
                o = jnp.dot(p, vb, preferred_element_type=jnp.float32)
                comm[0, idx * SQ:(idx + 1) * SQ, :] = o
                comm[0, L_ROW0:L_ROW0 + SQ, idx:idx + 1] = l

        for h in range(N_DEV - 1):
            rdma = pltpu.make_async_remote_copy(
                src_ref=comm.at[h],
                dst_ref=comm.at[h + 1],
                send_sem=send_sems.at[h],
                recv_sem=recv_sems.at[h],
                device_id=(right,),
                device_id_type=pl.DeviceIdType.MESH,
            )
            rdma.start()
            rdma.wait()

        for b in range(B):
            for h in range(HQ):
                idx = b * HQ + h
                r0 = idx * SQ
                o_tot = (comm[0, r0:r0 + SQ, :] + comm[1, r0:r0 + SQ, :]
                         + comm[2, r0:r0 + SQ, :] + comm[3, r0:r0 + SQ, :])
                l_tot = (comm[0, L_ROW0:L_ROW0 + SQ, idx:idx + 1]
                         + comm[1, L_ROW0:L_ROW0 + SQ, idx:idx + 1]
                         + comm[2, L_ROW0:L_ROW0 + SQ, idx:idx + 1]
                         + comm[3, L_ROW0:L_ROW0 + SQ, idx:idx + 1])
                att[b * SQ:(b + 1) * SQ, h * DH:(h + 1) * DH] = o_tot / l_tot

        out = jnp.dot(att[:], wo_ref[:], preferred_element_type=jnp.float32)
        out_ref[:] = out.reshape(B, SQ, D)

    return pl.pallas_call(
        body,
        out_shape=jax.ShapeDtypeStruct((B, SQ, D), jnp.float32),
        in_specs=[pl.BlockSpec(memory_space=pltpu.VMEM)] * 5,
        out_specs=pl.BlockSpec(memory_space=pltpu.VMEM),
        scratch_shapes=[
            pltpu.VMEM((N_DEV, ROWS, DH), jnp.float32),
            pltpu.VMEM((B * SQ, HQ * DH), jnp.float32),
            pltpu.SemaphoreType.DMA((N_DEV - 1,)),
            pltpu.SemaphoreType.DMA((N_DEV - 1,)),
        ],
        compiler_params=pltpu.CompilerParams(collective_id=0),
    )(x, Wq, Wo, K_ext, V_ext)


# device time: 96465 ns/iter; 1.9912x vs baseline; 1.9912x over previous
import jax
import jax.numpy as jnp
from jax import lax
from jax.experimental import pallas as pl
from jax.experimental.pallas import tpu as pltpu

N_DEV = 4
B, SQ, D = 4, 256, 1024
HQ, HKV, DH = 8, 2, 128
G = HQ // HKV
SCALE = 0.08838834764831843
NIDX = B * HQ
CQ = SQ // N_DEV
C_OROWS = NIDX * CQ
C_ROWS = C_OROWS + CQ
ROWS = N_DEV * C_ROWS


def kernel(x, Wq, Wo, K_ext, V_ext):
    def body(x_ref, wq_ref, wo_ref, k_ref, v_ref, out_ref,
             part, stage, att_q, out_q,
             rs_send, rs_recv, bc_send, bc_recv):
        my = lax.axis_index("i")

        barrier_sem = pltpu.get_barrier_semaphore()
        for nbr in ((my - 1) % N_DEV, (my + 1) % N_DEV):
            pl.semaphore_signal(barrier_sem, inc=1, device_id=(nbr,),
                                device_id_type=pl.DeviceIdType.MESH)
        pl.semaphore_wait(barrier_sem, 2)

        q = jnp.dot(x_ref[:].reshape(B * SQ, D), wq_ref[:],
                    preferred_element_type=jnp.float32)

        for b in range(B):
            for h in range(HQ):
                g = h // G
                idx = b * HQ + h
                qbh = q[b * SQ:(b + 1) * SQ, h * DH:(h + 1) * DH]
                kb = k_ref[b, :, g, :]
                vb = v_ref[b, :, g, :]
                s = lax.dot_general(
                    qbh, kb, (((1,), (1,)), ((), ())),
                    preferred_element_type=jnp.float32) * SCALE
                p = jnp.exp(s)
                l = jnp.sum(p, axis=1, keepdims=True)
                o = jnp.dot(p, vb, preferred_element_type=jnp.float32)
                for c in range(N_DEV):
                    r0 = c * C_ROWS
                    part[r0 + idx * CQ:r0 + (idx + 1) * CQ, :] = \
                        o[c * CQ:(c + 1) * CQ, :]
                    part[r0 + C_OROWS:r0 + C_ROWS, idx:idx + 1] = \
                        l[c * CQ:(c + 1) * CQ, :]

        rs = []
        for d in range(1, N_DEV):
            tgt = (my + d) % N_DEV
            rdma = pltpu.make_async_remote_copy(
                src_ref=part.at[pl.ds(tgt * C_ROWS, C_ROWS), :],
                dst_ref=stage.at[d - 1],
                send_sem=rs_send.at[d - 1],
                recv_sem=rs_recv.at[d - 1],
                device_id=(tgt,),
                device_id_type=pl.DeviceIdType.MESH,
            )
            rdma.start()
            rs.append(rdma)
        for rdma in rs:
            rdma.wait_recv()

        tot = (part[pl.ds(my * C_ROWS, C_ROWS), :]
               + stage[0] + stage[1] + stage[2])
        for b in range(B):
            for h in range(HQ):
                idx = b * HQ + h
                o_blk = tot[idx * CQ:(idx + 1) * CQ, :]
                l_blk = tot[C_OROWS:C_ROWS, idx:idx + 1]
                att_q[b * CQ:(b + 1) * CQ, h * DH:(h + 1) * DH] = o_blk / l_blk

        oq = jnp.dot(att_q[:], wo_ref[:], preferred_element_type=jnp.float32)
        for b in range(B):
            out_q[b] = oq[b * CQ:(b + 1) * CQ, :]

        out_ref[:, pl.ds(my * CQ, CQ), :] = out_q[:]
        bc = []
        for d in range(1, N_DEV):
            tgt = (my + d) % N_DEV
            rdma = pltpu.make_async_remote_copy(
                src_ref=out_q,
                dst_ref=out_ref.at[:, pl.ds(my * CQ, CQ), :],
                send_sem=bc_send.at[d - 1],
                recv_sem=bc_recv.at[d - 1],
                device_id=(tgt,),
                device_id_type=pl.DeviceIdType.MESH,
            )
            rdma.start()
            bc.append(rdma)
        for d in range(1, N_DEV):
            src_q = (my - d) % N_DEV
            pltpu.make_async_remote_copy(
                src_ref=out_q,
                dst_ref=out_ref.at[:, pl.ds(src_q * CQ, CQ), :],
                send_sem=bc_send.at[d - 1],
                recv_sem=bc_recv.at[d - 1],
                device_id=((my + d) % N_DEV,),
                device_id_type=pl.DeviceIdType.MESH,
            ).wait_recv()
        for rdma in rs:
            rdma.wait_send()
        for rdma in bc:
            rdma.wait_send()

    return pl.pallas_call(
        body,
        out_shape=jax.ShapeDtypeStruct((B, SQ, D), jnp.float32),
        in_specs=[pl.BlockSpec(memory_space=pltpu.VMEM)] * 5,
        out_specs=pl.BlockSpec(memory_space=pltpu.VMEM),
        scratch_shapes=[
            pltpu.VMEM((ROWS, DH), jnp.float32),
            pltpu.VMEM((N_DEV - 1, C_ROWS, DH), jnp.float32),
            pltpu.VMEM((B * CQ, HQ * DH), jnp.float32),
            pltpu.VMEM((B, CQ, D), jnp.float32),
            pltpu.SemaphoreType.DMA((N_DEV - 1,)),
            pltpu.SemaphoreType.DMA((N_DEV - 1,)),
            pltpu.SemaphoreType.DMA((N_DEV - 1,)),
            pltpu.SemaphoreType.DMA((N_DEV - 1,)),
        ],
        compiler_params=pltpu.CompilerParams(collective_id=0),
    )(x, Wq, Wo, K_ext, V_ext)


# device time: 74650 ns/iter; 2.5731x vs baseline; 1.2922x over previous
import jax
import jax.numpy as jnp
from jax import lax
from jax.experimental import pallas as pl
from jax.experimental.pallas import tpu as pltpu

N_DEV = 4
B, SQ, D = 4, 256, 1024
HQ, HKV, DH = 8, 2, 128
G = HQ // HKV
SCALE = 0.08838834764831843
C_OROWS = HQ * SQ
C_ROWS = C_OROWS + SQ
ROWS = N_DEV * C_ROWS
BF16 = jnp.bfloat16


def kernel(x, Wq, Wo, K_ext, V_ext):
    def body(x_ref, wq_ref, wo_ref, k_ref, v_ref, out_ref,
             part, stage, out_q,
             rs_send, rs_recv, bc_send, bc_recv):
        my = lax.axis_index("i")

        barrier_sem = pltpu.get_barrier_semaphore()
        for nbr in ((my - 1) % N_DEV, (my + 1) % N_DEV):
            pl.semaphore_signal(barrier_sem, inc=1, device_id=(nbr,),
                                device_id_type=pl.DeviceIdType.MESH)
        pl.semaphore_wait(barrier_sem, 2)

        x16 = x_ref[:].astype(BF16)
        wq16 = wq_ref[:].astype(BF16)
        k16 = k_ref[:].astype(BF16)
        v16 = v_ref[:].astype(BF16)

        for b in range(B):
            qb16 = jnp.dot(x16[b], wq16,
                           preferred_element_type=jnp.float32).astype(BF16)
            r0 = b * C_ROWS
            for h in range(HQ):
                g = h // G
                s = lax.dot_general(
                    qb16[:, h * DH:(h + 1) * DH], k16[b, :, g, :],
                    (((1,), (1,)), ((), ())),
                    preferred_element_type=jnp.float32) * SCALE
                p = jnp.exp(s)
                l = jnp.sum(p, axis=1, keepdims=True)
                o = jnp.dot(p.astype(BF16), v16[b, :, g, :],
                            preferred_element_type=jnp.float32)
                part[r0 + h * SQ:r0 + (h + 1) * SQ, :] = o.astype(BF16)
                part[r0 + C_OROWS:r0 + C_ROWS, h:h + 1] = l.astype(BF16)
            for d in range(1, N_DEV):
                @pl.when(((my + d) % N_DEV) == b)
                def _(b=b, d=d):
                    pltpu.make_async_remote_copy(
                        src_ref=part.at[pl.ds(b * C_ROWS, C_ROWS), :],
                        dst_ref=stage.at[d - 1],
                        send_sem=rs_send.at[d - 1],
                        recv_sem=rs_recv.at[d - 1],
                        device_id=(b,),
                        device_id_type=pl.DeviceIdType.MESH,
                    ).start()

        for j in range(N_DEV - 1):
            pltpu.make_async_remote_copy(
                src_ref=part.at[pl.ds(0, C_ROWS), :],
                dst_ref=stage.at[j],
                send_sem=rs_send.at[j],
                recv_sem=rs_recv.at[j],
                device_id=(my,),
                device_id_type=pl.DeviceIdType.MESH,
            ).wait_recv()

        tot = (part[pl.ds(my * C_ROWS, C_ROWS), :].astype(jnp.float32)
               + stage[0].astype(jnp.float32)
               + stage[1].astype(jnp.float32)
               + stage[2].astype(jnp.float32))
        cols = []
        for h in range(HQ):
            o_blk = tot[h * SQ:(h + 1) * SQ, :]
            l_blk = tot[C_OROWS:C_ROWS, h:h + 1]
            cols.append((o_blk / l_blk).astype(BF16))
        att16 = jnp.concatenate(cols, axis=1)
        wo16 = wo_ref[:].astype(BF16)
        oq = jnp.dot(att16, wo16, preferred_element_type=jnp.float32)
        out_q[0] = oq
        out_ref[pl.ds(my, 1)] = out_q[:]

        bc = []
        for d in range(1, N_DEV):
            rdma = pltpu.make_async_remote_copy(
                src_ref=out_q,
                dst_ref=out_ref.at[pl.ds(my, 1)],
                send_sem=bc_send.at[d - 1],
                recv_sem=bc_recv.at[d - 1],
                device_id=((my + d) % N_DEV,),
                device_id_type=pl.DeviceIdType.MESH,
            )
            rdma.start()
            bc.append(rdma)
        for j in range(N_DEV - 1):
            pltpu.make_async_remote_copy(
                src_ref=out_q,
                dst_ref=out_ref.at[pl.ds((my - j - 1) % N_DEV, 1)],
                send_sem=bc_send.at[j],
                recv_sem=bc_recv.at[j],
                device_id=((my + j + 1) % N_DEV,),
                device_id_type=pl.DeviceIdType.MESH,
            ).wait_recv()

        for j in range(N_DEV - 1):
            pltpu.make_async_remote_copy(
                src_ref=part.at[pl.ds(0, C_ROWS), :],
                dst_ref=stage.at[j],
                send_sem=rs_send.at[j],
                recv_sem=rs_recv.at[j],
                device_id=(my,),
                device_id_type=pl.DeviceIdType.MESH,
            ).wait_send()
        for rdma in bc:
            rdma.wait_send()

    return pl.pallas_call(
        body,
        out_shape=jax.ShapeDtypeStruct((B, SQ, D), jnp.float32),
        in_specs=[pl.BlockSpec(memory_space=pltpu.VMEM)] * 5,
        out_specs=pl.BlockSpec(memory_space=pltpu.VMEM),
        scratch_shapes=[
            pltpu.VMEM((ROWS, DH), BF16),
            pltpu.VMEM((N_DEV - 1, C_ROWS, DH), BF16),
            pltpu.VMEM((1, SQ, D), jnp.float32),
            pltpu.SemaphoreType.DMA((N_DEV - 1,)),
            pltpu.SemaphoreType.DMA((N_DEV - 1,)),
            pltpu.SemaphoreType.DMA((N_DEV - 1,)),
            pltpu.SemaphoreType.DMA((N_DEV - 1,)),
        ],
        compiler_params=pltpu.CompilerParams(
            collective_id=0, vmem_limit_bytes=100 * 1024 * 1024),
    )(x, Wq, Wo, K_ext, V_ext)
